# baseline (speedup 1.0000x reference)
import functools

import jax
import jax.numpy as jnp
from jax import lax
from jax.experimental import pallas as pl
from jax.experimental.pallas import tpu as pltpu
from jax.experimental.pallas import tpu_sc as plsc

N = 10000
E = 320000
D = 128
DE = 16

NC = 2
NS = 16
NW = NC * NS
EW = E // NW
C = 80
NCHUNK = EW // C
NBUF = 3
NIB = 2 * NBUF
NPAD = 10240
ROWS_PER_TILE = NPAD // NS
ZROWS = 8


def _emb_body(ae_ref, ao_ref, wt_ref, b_ref, out_ref):
    de = (jnp.dot(ae_ref[...], wt_ref[...], preferred_element_type=jnp.float32)
          + b_ref[...])
    do = (jnp.dot(ao_ref[...], wt_ref[...], preferred_element_type=jnp.float32)
          + b_ref[...])
    lo = lax.bitcast_convert_type(de.astype(jnp.bfloat16), jnp.uint16)
    hi = lax.bitcast_convert_type(do.astype(jnp.bfloat16), jnp.uint16)
    word = lo.astype(jnp.uint32) | (hi.astype(jnp.uint32) << 16)
    out_ref[...] = lax.bitcast_convert_type(word, jnp.float32)


def _edge_emb(edge_attr, wt, b2d):
    BEP = 2000
    EP = E // 2
    nb = EP // BEP
    return pl.pallas_call(
        _emb_body,
        grid=(nb,),
        in_specs=[
            pl.BlockSpec((BEP, DE), lambda i: (i, 0)),
            pl.BlockSpec((BEP, DE), lambda i: (i + nb, 0)),
            pl.BlockSpec((DE, D), lambda i: (0, 0)),
            pl.BlockSpec((1, D), lambda i: (0, 0)),
        ],
        out_specs=pl.BlockSpec((BEP, D), lambda i: (i, 0)),
        out_shape=jax.ShapeDtypeStruct((EP, D), jnp.float32),
    )(edge_attr, edge_attr, wt, b2d)


def _sc_kernel_body(src_hbm, dst_hbm, emb_hbm, h_hbm, out_hbm,
                    src_v, dst_v, emb_v, rows_v, stage_v, aggr_sh,
                    sem_src, sem_dst, sem_emb, sem_gat, sem_sct, sem_z):
    cid = lax.axis_index("c")
    sid = lax.axis_index("s")
    wid = sid * NC + cid
    pbase = wid * (EW // 2)
    PP = C // 2
    HALF = E // 2

    def zrow(r, carry):
        for j in range(D // 16):
            stage_v[r, pl.ds(j * 16, 16)] = jnp.zeros((16,), jnp.float32)
        return carry

    lax.fori_loop(0, ZROWS, zrow, 0)
    row_base = sid * ROWS_PER_TILE
    NZC = ROWS_PER_TILE // ZROWS
    def zcopy(t, carry):
        pltpu.async_copy(stage_v,
                         aggr_sh.at[pl.ds(row_base + t * ZROWS, ZROWS)], sem_z)
        return carry
    lax.fori_loop(0, NZC, zcopy, 0)
    def zwait(t, carry):
        pltpu.make_async_copy(stage_v, aggr_sh.at[pl.ds(row_base, ZROWS)],
                              sem_z).wait()
        return carry
    lax.fori_loop(0, NZC, zwait, 0)
    plsc.subcore_barrier()

    def start_idx(i, ib):
        pb = pbase + i * PP
        pltpu.async_copy(src_hbm.at[pl.ds(pb, PP)],
                         src_v.at[ib, pl.ds(0, PP)], sem_src.at[ib])
        pltpu.async_copy(src_hbm.at[pl.ds(HALF + pb, PP)],
                         src_v.at[ib, pl.ds(PP, PP)], sem_src.at[ib])
        pltpu.async_copy(dst_hbm.at[pl.ds(pb, PP)],
                         dst_v.at[ib, pl.ds(0, PP)], sem_dst.at[ib])
        pltpu.async_copy(dst_hbm.at[pl.ds(HALF + pb, PP)],
                         dst_v.at[ib, pl.ds(PP, PP)], sem_dst.at[ib])

    def wait_idx(i, ib):
        pb = pbase + i * PP
        pltpu.make_async_copy(src_hbm.at[pl.ds(pb, PP)],
                              src_v.at[ib, pl.ds(0, PP)],
                              sem_src.at[ib]).wait()
        pltpu.make_async_copy(src_hbm.at[pl.ds(HALF + pb, PP)],
                              src_v.at[ib, pl.ds(PP, PP)],
                              sem_src.at[ib]).wait()
        pltpu.make_async_copy(dst_hbm.at[pl.ds(pb, PP)],
                              dst_v.at[ib, pl.ds(0, PP)],
                              sem_dst.at[ib]).wait()
        pltpu.make_async_copy(dst_hbm.at[pl.ds(HALF + pb, PP)],
                              dst_v.at[ib, pl.ds(PP, PP)],
                              sem_dst.at[ib]).wait()

    def start_emb(i, b):
        pb = pbase + i * PP
        pltpu.async_copy(emb_hbm.at[pl.ds(pb, PP)], emb_v.at[b],
                         sem_emb.at[b])

    def start_gather(b, ib):
        pltpu.async_copy(h_hbm.at[src_v.at[ib]], rows_v.at[b], sem_gat.at[b])

    def wait_sct(b, ib):
        pltpu.make_async_copy(rows_v.at[b], aggr_sh.at[dst_v.at[ib]],
                              sem_sct.at[b]).wait()

    def unpack16(w):
        wi = lax.bitcast_convert_type(w, jnp.int32)
        lo = lax.bitcast_convert_type(lax.shift_left(wi, 16), jnp.float32)
        hi = lax.bitcast_convert_type(
            jnp.bitwise_and(wi, jnp.int32(-65536)), jnp.float32)
        return lo, hi

    def body(i, k, drain, do_prefetch, do_gather_next):
        b = k % NBUF
        b1 = (k + 1) % NBUF
        b2 = (k + 2) % NBUF
        ib1 = (k + 1) % NIB
        ib2 = (k + 2) % NIB
        ibd = (k + 4) % NIB
        pb = pbase + i * PP
        pltpu.make_async_copy(emb_hbm.at[pl.ds(pb, PP)], emb_v.at[b],
                              sem_emb.at[b]).wait()
        pltpu.make_async_copy(h_hbm.at[src_v.at[k % NIB]], rows_v.at[b],
                              sem_gat.at[b]).wait()

        @plsc.parallel_loop(0, C // 2, unroll=2)
        def pair(p):
            e1 = PP + p
            for j in range(D // 16):
                ea, eb = unpack16(emb_v[b, p, pl.ds(16 * j, 16)])
                s = pl.ds(16 * j, 16)
                rows_v[b, p, s] = jnp.maximum(rows_v[b, p, s] + ea, 0.0)
                rows_v[b, e1, s] = jnp.maximum(rows_v[b, e1, s] + eb, 0.0)
        pltpu.async_copy(rows_v.at[b], aggr_sh.at[dst_v.at[k % NIB]],
                         sem_sct.at[b], add=True)
        if do_prefetch:
            start_idx(i + 2, ib2)
            start_emb(i + 2, b2)
        if drain:
            wait_sct(b1, ibd)
        if do_gather_next:
            wait_idx(i + 1, ib1)
            start_gather(b1, ib1)

    start_idx(0, 0)
    start_emb(0, 0)
    start_idx(1, 1)
    start_emb(1, 1)
    wait_idx(0, 0)
    start_gather(0, 0)
    for i in range(NIB):
        body(i, i, drain=(i >= 2), do_prefetch=True, do_gather_next=True)

    LG = (NCHUNK - 5) // NIB

    def group(g, carry):
        i0 = g * NIB
        for k in range(NIB):
            body(i0 + k, k, drain=True, do_prefetch=True, do_gather_next=True)
        return carry

    lax.fori_loop(1, LG, group, 0)

    for i in range(NIB * LG, NCHUNK):
        body(i, i % NIB, drain=True,
             do_prefetch=(i + 2 <= NCHUNK - 1),
             do_gather_next=(i + 1 <= NCHUNK - 1))
    for j in range(NCHUNK - 2, NCHUNK):
        wait_sct(j % NBUF, j % NIB)
    plsc.subcore_barrier()

    pltpu.sync_copy(aggr_sh.at[pl.ds(row_base, ROWS_PER_TILE)],
                    out_hbm.at[cid, pl.ds(row_base, ROWS_PER_TILE)])


def _sc_aggregate(src, dst, emb, h):
    mesh = plsc.VectorSubcoreMesh(core_axis_name="c", subcore_axis_name="s")
    k = functools.partial(
        pl.kernel,
        mesh=mesh,
        compiler_params=pltpu.CompilerParams(),
        out_type=jax.ShapeDtypeStruct((NC, NPAD, D), jnp.float32),
        scratch_types=[
            pltpu.VMEM((NIB, C), jnp.int32),
            pltpu.VMEM((NIB, C), jnp.int32),
            pltpu.VMEM((NBUF, C // 2, D), jnp.float32),
            pltpu.VMEM((NBUF, C, D), jnp.float32),
            pltpu.VMEM((ZROWS, D), jnp.float32),
            pltpu.VMEM_SHARED((NPAD, D), jnp.float32),
            pltpu.SemaphoreType.DMA((NIB,)),
            pltpu.SemaphoreType.DMA((NIB,)),
            pltpu.SemaphoreType.DMA((NBUF,)),
            pltpu.SemaphoreType.DMA((NBUF,)),
            pltpu.SemaphoreType.DMA((NBUF,)),
            pltpu.SemaphoreType.DMA,
        ],
    )(_sc_kernel_body)
    return k(src, dst, emb, h)


def _epi_body(h_ref, a_ref, w1_ref, b1_ref, w2_ref, b2_ref, eps_ref, g_ref,
              bt_ref, out_ref):
    h = h_ref[...]
    x = (1.0 + eps_ref[0, 0]) * h + a_ref[0, :N] + a_ref[1, :N]
    y = jnp.maximum(
        jnp.dot(x, w1_ref[...], preferred_element_type=jnp.float32) + b1_ref[...],
        0.0,
    )
    y = jnp.dot(y, w2_ref[...], preferred_element_type=jnp.float32) + b2_ref[...]
    mean = jnp.mean(y, axis=0, keepdims=True)
    var = jnp.mean((y - mean) ** 2, axis=0, keepdims=True)
    out_ref[...] = g_ref[...] * (y - mean) * lax.rsqrt(var + 1e-5) + bt_ref[...] + h


def _epilogue(h, aggr, w1t, b1, w2t, b2, eps, gamma, beta):
    return pl.pallas_call(
        _epi_body,
        out_shape=jax.ShapeDtypeStruct((N, D), jnp.float32),
    )(h, aggr, w1t, b1.reshape(1, D), w2t, b2.reshape(1, D),
      eps.reshape(1, 1), gamma.reshape(1, D), beta.reshape(1, D))


def kernel(h, edge_index, edge_attr, lin_edge_W, lin_edge_b, mlp_W1, mlp_b1,
           mlp_W2, mlp_b2, eps, bn_gamma, bn_beta):
    src = edge_index[0].astype(jnp.int32)
    dst = edge_index[1].astype(jnp.int32)
    emb = _edge_emb(edge_attr, lin_edge_W.T, lin_edge_b.reshape(1, D))
    aggr = _sc_aggregate(src, dst, emb, h)
    return _epilogue(h, aggr, mlp_W1.T, mlp_b1, mlp_W2.T, mlp_b2, eps,
                     bn_gamma, bn_beta)


# speedup vs baseline: 1.0636x; 1.0636x over previous
import functools

import jax
import jax.numpy as jnp
from jax import lax
from jax.experimental import pallas as pl
from jax.experimental.pallas import tpu as pltpu
from jax.experimental.pallas import tpu_sc as plsc

N = 10000
E = 320000
D = 128
DE = 16

NC = 2
NS = 16
NW = NC * NS
EW = E // NW
C = 80
NCHUNK = EW // C
NBUF = 3
NIB = 2 * NBUF
NPAD = 10240
ROWS_PER_TILE = NPAD // NS
ZROWS = 8


def _emb_body(ae_ref, ao_ref, wt_ref, b_ref, out_ref):
    de = (jnp.dot(ae_ref[...], wt_ref[...], preferred_element_type=jnp.float32)
          + b_ref[...])
    do = (jnp.dot(ao_ref[...], wt_ref[...], preferred_element_type=jnp.float32)
          + b_ref[...])
    lo = lax.bitcast_convert_type(de.astype(jnp.bfloat16), jnp.uint16)
    hi = lax.bitcast_convert_type(do.astype(jnp.bfloat16), jnp.uint16)
    word = lo.astype(jnp.uint32) | (hi.astype(jnp.uint32) << 16)
    out_ref[...] = lax.bitcast_convert_type(word, jnp.float32)


def _edge_emb(edge_attr, wt, b2d):
    BEP = 4000
    EP = E // 2
    nb = EP // BEP
    return pl.pallas_call(
        _emb_body,
        grid=(nb,),
        in_specs=[
            pl.BlockSpec((BEP, DE), lambda i: (i, 0)),
            pl.BlockSpec((BEP, DE), lambda i: (i + nb, 0)),
            pl.BlockSpec((DE, D), lambda i: (0, 0)),
            pl.BlockSpec((1, D), lambda i: (0, 0)),
        ],
        out_specs=pl.BlockSpec((BEP, D), lambda i: (i, 0)),
        out_shape=jax.ShapeDtypeStruct((EP, D), jnp.float32),
    )(edge_attr, edge_attr, wt, b2d)


def _sc_kernel_body(src_hbm, dst_hbm, emb_hbm, h_hbm, out_hbm,
                    src_v, dst_v, emb_v, rows_v, stage_v, aggr_sh,
                    sem_src, sem_dst, sem_emb, sem_gat, sem_sct, sem_z):
    cid = lax.axis_index("c")
    sid = lax.axis_index("s")
    wid = sid * NC + cid
    pbase = wid * (EW // 2)
    PP = C // 2
    HALF = E // 2

    def zrow(r, carry):
        for j in range(D // 16):
            stage_v[r, pl.ds(j * 16, 16)] = jnp.zeros((16,), jnp.float32)
        return carry

    lax.fori_loop(0, ZROWS, zrow, 0)
    row_base = sid * ROWS_PER_TILE
    NZC = ROWS_PER_TILE // ZROWS
    def zcopy(t, carry):
        pltpu.async_copy(stage_v,
                         aggr_sh.at[pl.ds(row_base + t * ZROWS, ZROWS)], sem_z)
        return carry
    lax.fori_loop(0, NZC, zcopy, 0)
    def zwait(t, carry):
        pltpu.make_async_copy(stage_v, aggr_sh.at[pl.ds(row_base, ZROWS)],
                              sem_z).wait()
        return carry
    lax.fori_loop(0, NZC, zwait, 0)
    plsc.subcore_barrier()

    def start_idx(i, ib):
        pb = pbase + i * PP
        pltpu.async_copy(src_hbm.at[pl.ds(pb, PP)],
                         src_v.at[ib, pl.ds(0, PP)], sem_src.at[ib])
        pltpu.async_copy(src_hbm.at[pl.ds(HALF + pb, PP)],
                         src_v.at[ib, pl.ds(PP, PP)], sem_src.at[ib])
        pltpu.async_copy(dst_hbm.at[pl.ds(pb, PP)],
                         dst_v.at[ib, pl.ds(0, PP)], sem_dst.at[ib])
        pltpu.async_copy(dst_hbm.at[pl.ds(HALF + pb, PP)],
                         dst_v.at[ib, pl.ds(PP, PP)], sem_dst.at[ib])

    def wait_idx(i, ib):
        pb = pbase + i * PP
        pltpu.make_async_copy(src_hbm.at[pl.ds(pb, PP)],
                              src_v.at[ib, pl.ds(0, PP)],
                              sem_src.at[ib]).wait()
        pltpu.make_async_copy(src_hbm.at[pl.ds(HALF + pb, PP)],
                              src_v.at[ib, pl.ds(PP, PP)],
                              sem_src.at[ib]).wait()
        pltpu.make_async_copy(dst_hbm.at[pl.ds(pb, PP)],
                              dst_v.at[ib, pl.ds(0, PP)],
                              sem_dst.at[ib]).wait()
        pltpu.make_async_copy(dst_hbm.at[pl.ds(HALF + pb, PP)],
                              dst_v.at[ib, pl.ds(PP, PP)],
                              sem_dst.at[ib]).wait()

    def start_emb(i, b):
        pb = pbase + i * PP
        pltpu.async_copy(emb_hbm.at[pl.ds(pb, PP)], emb_v.at[b],
                         sem_emb.at[b])

    def start_gather(b, ib):
        pltpu.async_copy(h_hbm.at[src_v.at[ib]], rows_v.at[b], sem_gat.at[b])

    def wait_sct(b, ib):
        pltpu.make_async_copy(rows_v.at[b], aggr_sh.at[dst_v.at[ib]],
                              sem_sct.at[b]).wait()

    def unpack16(w):
        wi = lax.bitcast_convert_type(w, jnp.int32)
        lo = lax.bitcast_convert_type(lax.shift_left(wi, 16), jnp.float32)
        hi = lax.bitcast_convert_type(
            jnp.bitwise_and(wi, jnp.int32(-65536)), jnp.float32)
        return lo, hi

    def body(i, k, drain, do_prefetch, do_gather_next):
        b = k % NBUF
        b1 = (k + 1) % NBUF
        b2 = (k + 2) % NBUF
        ib1 = (k + 1) % NIB
        ib2 = (k + 2) % NIB
        ibd = (k + 4) % NIB
        pb = pbase + i * PP
        pltpu.make_async_copy(emb_hbm.at[pl.ds(pb, PP)], emb_v.at[b],
                              sem_emb.at[b]).wait()
        pltpu.make_async_copy(h_hbm.at[src_v.at[k % NIB]], rows_v.at[b],
                              sem_gat.at[b]).wait()

        def pair(p, c2):
            e1 = PP + p
            for j in range(D // 16):
                ea, eb = unpack16(emb_v[b, p, pl.ds(16 * j, 16)])
                s = pl.ds(16 * j, 16)
                rows_v[b, p, s] = jnp.maximum(rows_v[b, p, s] + ea, 0.0)
                rows_v[b, e1, s] = jnp.maximum(rows_v[b, e1, s] + eb, 0.0)
            return c2

        lax.fori_loop(0, C // 2, pair, 0)
        pltpu.async_copy(rows_v.at[b], aggr_sh.at[dst_v.at[k % NIB]],
                         sem_sct.at[b], add=True)
        if do_prefetch:
            start_idx(i + 2, ib2)
            start_emb(i + 2, b2)
        if drain:
            wait_sct(b1, ibd)
        if do_gather_next:
            wait_idx(i + 1, ib1)
            start_gather(b1, ib1)

    start_idx(0, 0)
    start_emb(0, 0)
    start_idx(1, 1)
    start_emb(1, 1)
    wait_idx(0, 0)
    start_gather(0, 0)
    for i in range(NIB):
        body(i, i, drain=(i >= 2), do_prefetch=True, do_gather_next=True)

    LG = (NCHUNK - 5) // NIB

    def group(g, carry):
        i0 = g * NIB
        for k in range(NIB):
            body(i0 + k, k, drain=True, do_prefetch=True, do_gather_next=True)
        return carry

    lax.fori_loop(1, LG, group, 0)

    for i in range(NIB * LG, NCHUNK):
        body(i, i % NIB, drain=True,
             do_prefetch=(i + 2 <= NCHUNK - 1),
             do_gather_next=(i + 1 <= NCHUNK - 1))
    for j in range(NCHUNK - 2, NCHUNK):
        wait_sct(j % NBUF, j % NIB)
    plsc.subcore_barrier()

    pltpu.sync_copy(aggr_sh.at[pl.ds(row_base, ROWS_PER_TILE)],
                    out_hbm.at[cid, pl.ds(row_base, ROWS_PER_TILE)])


def _sc_aggregate(src, dst, emb, h):
    mesh = plsc.VectorSubcoreMesh(core_axis_name="c", subcore_axis_name="s")
    k = functools.partial(
        pl.kernel,
        mesh=mesh,
        compiler_params=pltpu.CompilerParams(),
        out_type=jax.ShapeDtypeStruct((NC, NPAD, D), jnp.float32),
        scratch_types=[
            pltpu.VMEM((NIB, C), jnp.int32),
            pltpu.VMEM((NIB, C), jnp.int32),
            pltpu.VMEM((NBUF, C // 2, D), jnp.float32),
            pltpu.VMEM((NBUF, C, D), jnp.float32),
            pltpu.VMEM((ZROWS, D), jnp.float32),
            pltpu.VMEM_SHARED((NPAD, D), jnp.float32),
            pltpu.SemaphoreType.DMA((NIB,)),
            pltpu.SemaphoreType.DMA((NIB,)),
            pltpu.SemaphoreType.DMA((NBUF,)),
            pltpu.SemaphoreType.DMA((NBUF,)),
            pltpu.SemaphoreType.DMA((NBUF,)),
            pltpu.SemaphoreType.DMA,
        ],
    )(_sc_kernel_body)
    return k(src, dst, emb, h)


def _epi_body(h_ref, a_ref, w1_ref, b1_ref, w2_ref, b2_ref, eps_ref, g_ref,
              bt_ref, out_ref):
    h = h_ref[...]
    x = (1.0 + eps_ref[0, 0]) * h + a_ref[0, :N] + a_ref[1, :N]
    y = jnp.maximum(
        jnp.dot(x, w1_ref[...], preferred_element_type=jnp.float32) + b1_ref[...],
        0.0,
    )
    y = jnp.dot(y, w2_ref[...], preferred_element_type=jnp.float32) + b2_ref[...]
    mean = jnp.mean(y, axis=0, keepdims=True)
    var = jnp.mean((y - mean) ** 2, axis=0, keepdims=True)
    out_ref[...] = g_ref[...] * (y - mean) * lax.rsqrt(var + 1e-5) + bt_ref[...] + h


def _epilogue(h, aggr, w1t, b1, w2t, b2, eps, gamma, beta):
    return pl.pallas_call(
        _epi_body,
        out_shape=jax.ShapeDtypeStruct((N, D), jnp.float32),
    )(h, aggr, w1t, b1.reshape(1, D), w2t, b2.reshape(1, D),
      eps.reshape(1, 1), gamma.reshape(1, D), beta.reshape(1, D))


def kernel(h, edge_index, edge_attr, lin_edge_W, lin_edge_b, mlp_W1, mlp_b1,
           mlp_W2, mlp_b2, eps, bn_gamma, bn_beta):
    src = edge_index[0].astype(jnp.int32)
    dst = edge_index[1].astype(jnp.int32)
    emb = _edge_emb(edge_attr, lin_edge_W.T, lin_edge_b.reshape(1, D))
    aggr = _sc_aggregate(src, dst, emb, h)
    return _epilogue(h, aggr, mlp_W1.T, mlp_b1, mlp_W2.T, mlp_b2, eps,
                     bn_gamma, bn_beta)


# speedup vs baseline: 1.0639x; 1.0003x over previous
import functools

import jax
import jax.numpy as jnp
from jax import lax
from jax.experimental import pallas as pl
from jax.experimental.pallas import tpu as pltpu
from jax.experimental.pallas import tpu_sc as plsc

N = 10000
E = 320000
D = 128
DE = 16

NC = 2
NS = 16
NW = NC * NS
EW = E // NW
C = 80
NCHUNK = EW // C
NBUF = 3
NIB = 2 * NBUF
NPAD = 10240
ROWS_PER_TILE = NPAD // NS
ZROWS = 8


def _emb_body(ae_ref, ao_ref, wt_ref, b_ref, out_ref):
    de = (jnp.dot(ae_ref[...], wt_ref[...], preferred_element_type=jnp.float32)
          + b_ref[...])
    do = (jnp.dot(ao_ref[...], wt_ref[...], preferred_element_type=jnp.float32)
          + b_ref[...])
    lo = lax.bitcast_convert_type(de.astype(jnp.bfloat16), jnp.uint16)
    hi = lax.bitcast_convert_type(do.astype(jnp.bfloat16), jnp.uint16)
    word = lo.astype(jnp.uint32) | (hi.astype(jnp.uint32) << 16)
    out_ref[...] = lax.bitcast_convert_type(word, jnp.float32)


def _edge_emb(edge_attr, wt, b2d):
    BEP = 4000
    EP = E // 2
    nb = EP // BEP
    return pl.pallas_call(
        _emb_body,
        grid=(nb,),
        in_specs=[
            pl.BlockSpec((BEP, DE), lambda i: (i, 0)),
            pl.BlockSpec((BEP, DE), lambda i: (i + nb, 0)),
            pl.BlockSpec((DE, D), lambda i: (0, 0)),
            pl.BlockSpec((1, D), lambda i: (0, 0)),
        ],
        out_specs=pl.BlockSpec((BEP, D), lambda i: (i, 0)),
        out_shape=jax.ShapeDtypeStruct((EP, D), jnp.float32),
    )(edge_attr, edge_attr, wt, b2d)


def _sc_kernel_body(src_hbm, dst_hbm, emb_hbm, h_hbm, out_hbm,
                    src_v, dst_v, emb_v, rows_v, stage_v, aggr_sh,
                    sem_src, sem_dst, sem_emb, sem_gat, sem_sct, sem_z):
    cid = lax.axis_index("c")
    sid = lax.axis_index("s")
    wid = sid * NC + cid
    pbase = wid * (EW // 2)
    PP = C // 2
    HALF = E // 2

    def zrow(r, carry):
        for j in range(D // 16):
            stage_v[r, pl.ds(j * 16, 16)] = jnp.zeros((16,), jnp.float32)
        return carry

    lax.fori_loop(0, ZROWS, zrow, 0)
    row_base = sid * ROWS_PER_TILE
    NZC = ROWS_PER_TILE // ZROWS
    def zcopy(t, carry):
        pltpu.async_copy(stage_v,
                         aggr_sh.at[pl.ds(row_base + t * ZROWS, ZROWS)], sem_z)
        return carry
    lax.fori_loop(0, NZC, zcopy, 0)
    def zwait(t, carry):
        pltpu.make_async_copy(stage_v, aggr_sh.at[pl.ds(row_base, ZROWS)],
                              sem_z).wait()
        return carry
    lax.fori_loop(0, NZC, zwait, 0)
    plsc.subcore_barrier()

    def start_idx(i, ib):
        pb = pbase + i * PP
        pltpu.async_copy(src_hbm.at[pl.ds(pb, PP)],
                         src_v.at[ib, pl.ds(0, PP)], sem_src.at[ib])
        pltpu.async_copy(src_hbm.at[pl.ds(HALF + pb, PP)],
                         src_v.at[ib, pl.ds(PP, PP)], sem_src.at[ib])
        pltpu.async_copy(dst_hbm.at[pl.ds(pb, PP)],
                         dst_v.at[ib, pl.ds(0, PP)], sem_dst.at[ib])
        pltpu.async_copy(dst_hbm.at[pl.ds(HALF + pb, PP)],
                         dst_v.at[ib, pl.ds(PP, PP)], sem_dst.at[ib])

    def wait_idx(i, ib):
        pb = pbase + i * PP
        pltpu.make_async_copy(src_hbm.at[pl.ds(pb, PP)],
                              src_v.at[ib, pl.ds(0, PP)],
                              sem_src.at[ib]).wait()
        pltpu.make_async_copy(src_hbm.at[pl.ds(HALF + pb, PP)],
                              src_v.at[ib, pl.ds(PP, PP)],
                              sem_src.at[ib]).wait()
        pltpu.make_async_copy(dst_hbm.at[pl.ds(pb, PP)],
                              dst_v.at[ib, pl.ds(0, PP)],
                              sem_dst.at[ib]).wait()
        pltpu.make_async_copy(dst_hbm.at[pl.ds(HALF + pb, PP)],
                              dst_v.at[ib, pl.ds(PP, PP)],
                              sem_dst.at[ib]).wait()

    def start_emb(i, b):
        pb = pbase + i * PP
        pltpu.async_copy(emb_hbm.at[pl.ds(pb, PP)], emb_v.at[b],
                         sem_emb.at[b])

    def start_gather(b, ib):
        pltpu.async_copy(h_hbm.at[src_v.at[ib]], rows_v.at[b], sem_gat.at[b])

    def wait_sct(b, ib):
        pltpu.make_async_copy(rows_v.at[b], aggr_sh.at[dst_v.at[ib]],
                              sem_sct.at[b]).wait()

    def unpack16(w):
        wi = lax.bitcast_convert_type(w, jnp.int32)
        lo = lax.bitcast_convert_type(lax.shift_left(wi, 16), jnp.float32)
        hi = lax.bitcast_convert_type(
            jnp.bitwise_and(wi, jnp.int32(-65536)), jnp.float32)
        return lo, hi

    def body(i, k, drain, do_prefetch, do_gather_next):
        b = k % NBUF
        b1 = (k + 1) % NBUF
        b2 = (k + 2) % NBUF
        ib1 = (k + 1) % NIB
        ib2 = (k + 2) % NIB
        ibd = (k + 4) % NIB
        pb = pbase + i * PP
        pltpu.make_async_copy(emb_hbm.at[pl.ds(pb, PP)], emb_v.at[b],
                              sem_emb.at[b]).wait()
        pltpu.make_async_copy(h_hbm.at[src_v.at[k % NIB]], rows_v.at[b],
                              sem_gat.at[b]).wait()

        def pair(p, c2):
            e1 = PP + p
            for j in range(D // 16):
                ea, eb = unpack16(emb_v[b, p, pl.ds(16 * j, 16)])
                s = pl.ds(16 * j, 16)
                rows_v[b, p, s] = jnp.maximum(rows_v[b, p, s] + ea, 0.0)
                rows_v[b, e1, s] = jnp.maximum(rows_v[b, e1, s] + eb, 0.0)
            return c2

        lax.fori_loop(0, C // 2, pair, 0)
        pltpu.async_copy(rows_v.at[b], aggr_sh.at[dst_v.at[k % NIB]],
                         sem_sct.at[b], add=True)
        if do_prefetch:
            start_idx(i + 2, ib2)
            start_emb(i + 2, b2)
        if drain:
            wait_sct(b1, ibd)
        if do_gather_next:
            wait_idx(i + 1, ib1)
            start_gather(b1, ib1)

    start_idx(0, 0)
    start_emb(0, 0)
    start_idx(1, 1)
    start_emb(1, 1)
    wait_idx(0, 0)
    start_gather(0, 0)
    for i in range(NIB):
        body(i, i, drain=(i >= 2), do_prefetch=True, do_gather_next=True)

    LG = (NCHUNK - 5) // NIB

    def group(g, carry):
        i0 = g * NIB
        for k in range(NIB):
            body(i0 + k, k, drain=True, do_prefetch=True, do_gather_next=True)
        return carry

    lax.fori_loop(1, LG, group, 0)

    for i in range(NIB * LG, NCHUNK):
        body(i, i % NIB, drain=True,
             do_prefetch=(i + 2 <= NCHUNK - 1),
             do_gather_next=(i + 1 <= NCHUNK - 1))
    for j in range(NCHUNK - 2, NCHUNK):
        wait_sct(j % NBUF, j % NIB)
    plsc.subcore_barrier()

    pltpu.sync_copy(aggr_sh.at[pl.ds(row_base, ROWS_PER_TILE)],
                    out_hbm.at[cid, pl.ds(row_base, ROWS_PER_TILE)])


def _sc_aggregate(src, dst, emb, h):
    mesh = plsc.VectorSubcoreMesh(core_axis_name="c", subcore_axis_name="s")
    k = functools.partial(
        pl.kernel,
        mesh=mesh,
        compiler_params=pltpu.CompilerParams(skip_device_barrier=True),
        out_type=jax.ShapeDtypeStruct((NC, NPAD, D), jnp.float32),
        scratch_types=[
            pltpu.VMEM((NIB, C), jnp.int32),
            pltpu.VMEM((NIB, C), jnp.int32),
            pltpu.VMEM((NBUF, C // 2, D), jnp.float32),
            pltpu.VMEM((NBUF, C, D), jnp.float32),
            pltpu.VMEM((ZROWS, D), jnp.float32),
            pltpu.VMEM_SHARED((NPAD, D), jnp.float32),
            pltpu.SemaphoreType.DMA((NIB,)),
            pltpu.SemaphoreType.DMA((NIB,)),
            pltpu.SemaphoreType.DMA((NBUF,)),
            pltpu.SemaphoreType.DMA((NBUF,)),
            pltpu.SemaphoreType.DMA((NBUF,)),
            pltpu.SemaphoreType.DMA,
        ],
    )(_sc_kernel_body)
    return k(src, dst, emb, h)


def _epi_body(h_ref, a_ref, w1_ref, b1_ref, w2_ref, b2_ref, eps_ref, g_ref,
              bt_ref, out_ref):
    h = h_ref[...]
    x = (1.0 + eps_ref[0, 0]) * h + a_ref[0, :N] + a_ref[1, :N]
    y = jnp.maximum(
        jnp.dot(x, w1_ref[...], preferred_element_type=jnp.float32) + b1_ref[...],
        0.0,
    )
    y = jnp.dot(y, w2_ref[...], preferred_element_type=jnp.float32) + b2_ref[...]
    mean = jnp.mean(y, axis=0, keepdims=True)
    var = jnp.mean((y - mean) ** 2, axis=0, keepdims=True)
    out_ref[...] = g_ref[...] * (y - mean) * lax.rsqrt(var + 1e-5) + bt_ref[...] + h


def _epilogue(h, aggr, w1t, b1, w2t, b2, eps, gamma, beta):
    return pl.pallas_call(
        _epi_body,
        out_shape=jax.ShapeDtypeStruct((N, D), jnp.float32),
    )(h, aggr, w1t, b1.reshape(1, D), w2t, b2.reshape(1, D),
      eps.reshape(1, 1), gamma.reshape(1, D), beta.reshape(1, D))


def kernel(h, edge_index, edge_attr, lin_edge_W, lin_edge_b, mlp_W1, mlp_b1,
           mlp_W2, mlp_b2, eps, bn_gamma, bn_beta):
    src = edge_index[0].astype(jnp.int32)
    dst = edge_index[1].astype(jnp.int32)
    emb = _edge_emb(edge_attr, lin_edge_W.T, lin_edge_b.reshape(1, D))
    aggr = _sc_aggregate(src, dst, emb, h)
    return _epilogue(h, aggr, mlp_W1.T, mlp_b1, mlp_W2.T, mlp_b2, eps,
                     bn_gamma, bn_beta)
